# Initial kernel scaffold; baseline (speedup 1.0000x reference)
import jax
import jax.numpy as jnp
from jax.experimental import pallas as pl


def kernel(x, Wq, bq, Wkv, bkv, Wp, bp):
    raise NotImplementedError("write your pallas kernel here")


# speedup vs baseline: 23.2078x; 23.2078x over previous
import functools

import jax
import jax.numpy as jnp
from jax.experimental import pallas as pl

DIM = 384
NUM_HEADS = 8
TOP_K = 16
HEAD_DIM = DIM // NUM_HEADS
SCALE = HEAD_DIM ** (-0.5)
NEG = -1e30


def _fused_attn_kernel(x_ref, wq_ref, bq_ref, wk_ref, bk_ref, wv_ref, bv_ref,
                       wp_ref, bp_ref, out_ref):
    h = pl.program_id(1)
    xb = x_ref[0]
    q = jnp.dot(xb, wq_ref[0], preferred_element_type=jnp.float32) + bq_ref[0]
    q = jnp.maximum(q, 0.0)
    k = jnp.dot(xb, wk_ref[0], preferred_element_type=jnp.float32) + bk_ref[0]
    v = jnp.dot(xb, wv_ref[0], preferred_element_type=jnp.float32) + bv_ref[0]

    attn = jax.lax.dot_general(
        q, k, (((1,), (1,)), ((), ())),
        preferred_element_type=jnp.float32) * SCALE

    work = attn
    for _ in range(TOP_K - 1):
        m = jnp.max(work, axis=1, keepdims=True)
        work = jnp.where(work >= m, NEG, work)
    thresh = jnp.max(work, axis=1, keepdims=True)

    p = jnp.exp(jnp.where(attn >= thresh, attn, 0.0)) - 1.0
    n = jnp.float32(attn.shape[1])
    denom = jnp.sum(p, axis=1, keepdims=True) + n
    num = (jnp.dot(p, v, preferred_element_type=jnp.float32)
           + jnp.sum(v, axis=0, keepdims=True))
    attnout = num / denom

    partial = jnp.dot(attnout, wp_ref[...],
                      preferred_element_type=jnp.float32)

    @pl.when(h == 0)
    def _():
        out_ref[0] = partial + bp_ref[0]

    @pl.when(h != 0)
    def _():
        out_ref[0] = out_ref[0] + partial


@jax.jit
def kernel(x, Wq, bq, Wkv, bkv, Wp, bp):
    B, N, C = x.shape
    H, dh = NUM_HEADS, HEAD_DIM
    Wq_r = Wq.reshape(C, H, dh).transpose(1, 0, 2)
    Wk_r = Wkv[:, :C].reshape(C, H, dh).transpose(1, 0, 2)
    Wv_r = Wkv[:, C:].reshape(C, H, dh).transpose(1, 0, 2)
    bq_r = bq.reshape(H, 1, dh)
    bk_r = bkv[:C].reshape(H, 1, dh)
    bv_r = bkv[C:].reshape(H, 1, dh)
    bp_r = bp.reshape(1, C)

    grid = (B, H)
    out = pl.pallas_call(
        _fused_attn_kernel,
        grid=grid,
        in_specs=[
            pl.BlockSpec((1, N, C), lambda b, h: (b, 0, 0)),
            pl.BlockSpec((1, C, dh), lambda b, h: (h, 0, 0)),
            pl.BlockSpec((1, 1, dh), lambda b, h: (h, 0, 0)),
            pl.BlockSpec((1, C, dh), lambda b, h: (h, 0, 0)),
            pl.BlockSpec((1, 1, dh), lambda b, h: (h, 0, 0)),
            pl.BlockSpec((1, C, dh), lambda b, h: (h, 0, 0)),
            pl.BlockSpec((1, 1, dh), lambda b, h: (h, 0, 0)),
            pl.BlockSpec((dh, C), lambda b, h: (h, 0)),
            pl.BlockSpec((1, C), lambda b, h: (0, 0)),
        ],
        out_specs=pl.BlockSpec((1, N, C), lambda b, h: (b, 0, 0)),
        out_shape=jax.ShapeDtypeStruct((B, N, C), jnp.float32),
    )(x, Wq_r, bq_r, Wk_r, bk_r, Wv_r, bv_r, Wp, bp_r)
    return out


# speedup vs baseline: 41.1403x; 1.7727x over previous
import jax
import jax.numpy as jnp
from jax.experimental import pallas as pl

DIM = 384
NUM_HEADS = 8
TOP_K = 16
HEAD_DIM = DIM // NUM_HEADS
SCALE = HEAD_DIM ** (-0.5)

_SORT8 = [(0, 1), (2, 3), (4, 5), (6, 7), (0, 2), (1, 3), (4, 6), (5, 7),
          (1, 2), (5, 6), (0, 4), (1, 5), (2, 6), (3, 7), (2, 4), (3, 5),
          (1, 2), (3, 4), (5, 6)]


def _ce(slots, i, j):
    a, b = slots[i], slots[j]
    slots[i] = jnp.maximum(a, b)
    slots[j] = jnp.minimum(a, b)


def _bitonic_resort_desc(slots):
    n = len(slots)
    stride = n // 2
    while stride >= 1:
        for i in range(n):
            if (i % (2 * stride)) < stride:
                _ce(slots, i, i + stride)
        stride //= 2
    return slots


def _top16_threshold(at):
    rows = at.shape[0]
    slots = [at[128 * s:128 * (s + 1), :] for s in range(8)]
    for i, j in _SORT8:
        _ce(slots, i, j)
    half = rows // 16
    x = [s[:half] for s in slots]
    y = [s[half:] for s in slots]
    hi = [jnp.maximum(x[i], y[7 - i]) for i in range(8)]
    lo = [jnp.minimum(x[i], y[7 - i]) for i in range(8)]
    t = _bitonic_resort_desc(hi) + _bitonic_resort_desc(lo)
    n = half
    while n > 1:
        m = n // 2
        x = [s[:m] for s in t]
        y = [s[m:] for s in t]
        t = _bitonic_resort_desc(
            [jnp.maximum(x[i], y[15 - i]) for i in range(16)])
        n = m
    return t[15]


def _fused_attn_kernel(x_ref, wq_ref, bq_ref, wk_ref, bk_ref, wv_ref, bv_ref,
                       wp_ref, bp_ref, out_ref):
    h = pl.program_id(1)
    xb = x_ref[0]
    q = jnp.dot(xb, wq_ref[0], preferred_element_type=jnp.float32) + bq_ref[0]
    q = jnp.maximum(q, 0.0)
    k = jnp.dot(xb, wk_ref[0], preferred_element_type=jnp.float32) + bk_ref[0]
    v = jnp.dot(xb, wv_ref[0], preferred_element_type=jnp.float32) + bv_ref[0]

    at = jax.lax.dot_general(
        k, q, (((1,), (1,)), ((), ())),
        preferred_element_type=jnp.float32) * SCALE

    thresh = _top16_threshold(at)

    p = jnp.exp(jnp.where(at >= thresh, at, 0.0)) - 1.0

    n = at.shape[0]
    v1 = jnp.concatenate([v, jnp.ones((n, 1), jnp.float32)], axis=1)
    ap = jax.lax.dot_general(
        p, v1, (((0,), (0,)), ((), ())),
        preferred_element_type=jnp.float32)
    ap = ap + jnp.sum(v1, axis=0, keepdims=True)
    attnout = ap[:, :HEAD_DIM] / ap[:, HEAD_DIM:HEAD_DIM + 1]

    partial = jnp.dot(attnout, wp_ref[...],
                      preferred_element_type=jnp.float32)

    @pl.when(h == 0)
    def _():
        out_ref[0] = partial + bp_ref[0]

    @pl.when(h != 0)
    def _():
        out_ref[0] = out_ref[0] + partial


@jax.jit
def kernel(x, Wq, bq, Wkv, bkv, Wp, bp):
    B, N, C = x.shape
    H, dh = NUM_HEADS, HEAD_DIM
    Wq_r = Wq.reshape(C, H, dh).transpose(1, 0, 2)
    Wk_r = Wkv[:, :C].reshape(C, H, dh).transpose(1, 0, 2)
    Wv_r = Wkv[:, C:].reshape(C, H, dh).transpose(1, 0, 2)
    bq_r = bq.reshape(H, 1, dh)
    bk_r = bkv[:C].reshape(H, 1, dh)
    bv_r = bkv[C:].reshape(H, 1, dh)
    bp_r = bp.reshape(1, C)

    grid = (B, H)
    out = pl.pallas_call(
        _fused_attn_kernel,
        grid=grid,
        in_specs=[
            pl.BlockSpec((1, N, C), lambda b, h: (b, 0, 0)),
            pl.BlockSpec((1, C, dh), lambda b, h: (h, 0, 0)),
            pl.BlockSpec((1, 1, dh), lambda b, h: (h, 0, 0)),
            pl.BlockSpec((1, C, dh), lambda b, h: (h, 0, 0)),
            pl.BlockSpec((1, 1, dh), lambda b, h: (h, 0, 0)),
            pl.BlockSpec((1, C, dh), lambda b, h: (h, 0, 0)),
            pl.BlockSpec((1, 1, dh), lambda b, h: (h, 0, 0)),
            pl.BlockSpec((dh, C), lambda b, h: (h, 0)),
            pl.BlockSpec((1, C), lambda b, h: (0, 0)),
        ],
        out_specs=pl.BlockSpec((1, N, C), lambda b, h: (b, 0, 0)),
        out_shape=jax.ShapeDtypeStruct((B, N, C), jnp.float32),
    )(x, Wq_r, bq_r, Wk_r, bk_r, Wv_r, bv_r, Wp, bp_r)
    return out


# speedup vs baseline: 43.8390x; 1.0656x over previous
import jax
import jax.numpy as jnp
from jax.experimental import pallas as pl

DIM = 384
NUM_HEADS = 8
TOP_K = 16
HEAD_DIM = DIM // NUM_HEADS
SCALE = HEAD_DIM ** (-0.5)

_SORT16 = [
    (0, 1), (2, 3), (0, 2), (1, 3), (1, 2), (4, 5), (6, 7), (4, 6), (5, 7),
    (5, 6), (0, 4), (2, 6), (2, 4), (1, 5), (3, 7), (3, 5), (1, 2), (3, 4),
    (5, 6), (8, 9), (10, 11), (8, 10), (9, 11), (9, 10), (12, 13), (14, 15),
    (12, 14), (13, 15), (13, 14), (8, 12), (10, 14), (10, 12), (9, 13),
    (11, 15), (11, 13), (9, 10), (11, 12), (13, 14), (0, 8), (4, 12), (4, 8),
    (2, 10), (6, 14), (6, 10), (2, 4), (6, 8), (10, 12), (1, 9), (5, 13),
    (5, 9), (3, 11), (7, 15), (7, 11), (3, 5), (7, 9), (11, 13), (1, 2),
    (3, 4), (5, 6), (7, 8), (9, 10), (11, 12), (13, 14)]


def _ce(slots, i, j):
    a, b = slots[i], slots[j]
    slots[i] = jnp.maximum(a, b)
    slots[j] = jnp.minimum(a, b)


def _bitonic_resort_desc(slots):
    n = len(slots)
    stride = n // 2
    while stride >= 1:
        for i in range(n):
            if (i % (2 * stride)) < stride:
                _ce(slots, i, i + stride)
        stride //= 2
    return slots


def _top16_threshold(at):
    rows = at.shape[0]
    runs = rows // 16
    t = [at[runs * s:runs * (s + 1), :] for s in range(16)]
    for i, j in _SORT16:
        _ce(t, i, j)
    n = runs
    while n > 1:
        m = n // 2
        x = [s[:m] for s in t]
        y = [s[m:] for s in t]
        t = _bitonic_resort_desc(
            [jnp.maximum(x[i], y[15 - i]) for i in range(16)])
        n = m
    return t[15]


def _fused_attn_kernel(x_ref, xt_ref, wqt_ref, bqt_ref, wk_ref, bk_ref,
                       wvt_ref, bvt_ref, wp_ref, bp_ref, out_ref):
    h = pl.program_id(1)
    xb = x_ref[0]
    xtb = xt_ref[0]
    qt = jnp.dot(wqt_ref[0], xtb, preferred_element_type=jnp.float32) \
        + bqt_ref[0]
    qt = jnp.maximum(qt, 0.0)
    k = jnp.dot(xb, wk_ref[0], preferred_element_type=jnp.float32) + bk_ref[0]
    vt = jnp.dot(wvt_ref[0], xtb, preferred_element_type=jnp.float32) \
        + bvt_ref[0]

    at = jax.lax.dot_general(
        k, qt, (((1,), (0,)), ((), ())),
        preferred_element_type=jnp.float32) * SCALE

    thresh = _top16_threshold(at)

    p = jnp.exp(jnp.where(at >= thresh, at, 0.0))

    n = at.shape[0]
    v1t = jnp.concatenate([vt, jnp.ones((1, n), jnp.float32)], axis=0)
    apt = jax.lax.dot_general(
        v1t, p, (((1,), (0,)), ((), ())),
        preferred_element_type=jnp.float32)
    attnout_t = apt[:HEAD_DIM] / apt[HEAD_DIM:HEAD_DIM + 1]

    partial = jax.lax.dot_general(
        attnout_t, wp_ref[...], (((0,), (0,)), ((), ())),
        preferred_element_type=jnp.float32)

    @pl.when(h == 0)
    def _():
        out_ref[0] = partial + bp_ref[0]

    @pl.when(h != 0)
    def _():
        out_ref[0] = out_ref[0] + partial


@jax.jit
def kernel(x, Wq, bq, Wkv, bkv, Wp, bp):
    B, N, C = x.shape
    H, dh = NUM_HEADS, HEAD_DIM
    xt = x.transpose(0, 2, 1)
    Wqt_r = Wq.reshape(C, H, dh).transpose(1, 2, 0)
    Wk_r = Wkv[:, :C].reshape(C, H, dh).transpose(1, 0, 2)
    Wvt_r = Wkv[:, C:].reshape(C, H, dh).transpose(1, 2, 0)
    bqt_r = bq.reshape(H, dh, 1)
    bk_r = bkv[:C].reshape(H, 1, dh)
    bvt_r = bkv[C:].reshape(H, dh, 1)
    bp_r = bp.reshape(1, C)

    grid = (B, H)
    out = pl.pallas_call(
        _fused_attn_kernel,
        grid=grid,
        in_specs=[
            pl.BlockSpec((1, N, C), lambda b, h: (b, 0, 0)),
            pl.BlockSpec((1, C, N), lambda b, h: (b, 0, 0)),
            pl.BlockSpec((1, dh, C), lambda b, h: (h, 0, 0)),
            pl.BlockSpec((1, dh, 1), lambda b, h: (h, 0, 0)),
            pl.BlockSpec((1, C, dh), lambda b, h: (h, 0, 0)),
            pl.BlockSpec((1, 1, dh), lambda b, h: (h, 0, 0)),
            pl.BlockSpec((1, dh, C), lambda b, h: (h, 0, 0)),
            pl.BlockSpec((1, dh, 1), lambda b, h: (h, 0, 0)),
            pl.BlockSpec((dh, C), lambda b, h: (h, 0)),
            pl.BlockSpec((1, C), lambda b, h: (0, 0)),
        ],
        out_specs=pl.BlockSpec((1, N, C), lambda b, h: (b, 0, 0)),
        out_shape=jax.ShapeDtypeStruct((B, N, C), jnp.float32),
    )(x, xt, Wqt_r, bqt_r, Wk_r, bk_r, Wvt_r, bvt_r, Wp, bp_r)
    return out


# speedup vs baseline: 49.3573x; 1.1259x over previous
import jax
import jax.numpy as jnp
from jax.experimental import pallas as pl
from jax.experimental.pallas import tpu as pltpu

DIM = 384
NUM_HEADS = 8
TOP_K = 16
HEAD_DIM = DIM // NUM_HEADS
SCALE = HEAD_DIM ** (-0.5)

_SORT16 = [
    (0, 1), (2, 3), (0, 2), (1, 3), (1, 2), (4, 5), (6, 7), (4, 6), (5, 7),
    (5, 6), (0, 4), (2, 6), (2, 4), (1, 5), (3, 7), (3, 5), (1, 2), (3, 4),
    (5, 6), (8, 9), (10, 11), (8, 10), (9, 11), (9, 10), (12, 13), (14, 15),
    (12, 14), (13, 15), (13, 14), (8, 12), (10, 14), (10, 12), (9, 13),
    (11, 15), (11, 13), (9, 10), (11, 12), (13, 14), (0, 8), (4, 12), (4, 8),
    (2, 10), (6, 14), (6, 10), (2, 4), (6, 8), (10, 12), (1, 9), (5, 13),
    (5, 9), (3, 11), (7, 15), (7, 11), (3, 5), (7, 9), (11, 13), (1, 2),
    (3, 4), (5, 6), (7, 8), (9, 10), (11, 12), (13, 14)]


def _ce(slots, i, j):
    a, b = slots[i], slots[j]
    slots[i] = jnp.maximum(a, b)
    slots[j] = jnp.minimum(a, b)


def _bitonic_resort_desc(slots):
    n = len(slots)
    stride = n // 2
    while stride >= 1:
        for i in range(n):
            if (i % (2 * stride)) < stride:
                _ce(slots, i, i + stride)
        stride //= 2
    return slots


def _topk_weights(at):
    rows, cols = at.shape
    runs = rows // 16
    parts = []
    for c0 in range(0, cols, 128):
        ch = at[:, c0:c0 + 128]
        t = [ch[runs * s:runs * (s + 1), :] for s in range(16)]
        for i, j in _SORT16:
            _ce(t, i, j)
        n = runs
        while n > 2:
            m = n // 2
            x = [s[:m] for s in t]
            y = [s[m:] for s in t]
            t = _bitonic_resort_desc(
                [jnp.maximum(x[i], y[15 - i]) for i in range(16)])
            n = m
        z = [jnp.maximum(t[i][:1], t[15 - i][1:]) for i in range(16)]
        thr = z[0]
        for zi in z[1:]:
            thr = jnp.minimum(thr, zi)
        parts.append(jnp.exp(jnp.where(ch >= thr, ch, 0.0)))
    return jnp.concatenate(parts, axis=1)


def _fused_attn_kernel(x_ref, xt_ref, wqt_ref, bqt_ref, wk_ref, bk_ref,
                       wvt_ref, bvt_ref, wpt_ref, bp_ref, out_ref, ao_ref):
    h = pl.program_id(1)
    xb = x_ref[0]
    xtb = xt_ref[0]
    qt = jnp.dot(wqt_ref[0], xtb, preferred_element_type=jnp.float32) \
        + bqt_ref[0]
    qt = jnp.maximum(qt, 0.0)
    k = jnp.dot(xb, wk_ref[0], preferred_element_type=jnp.float32) + bk_ref[0]
    vt = jnp.dot(wvt_ref[0], xtb, preferred_element_type=jnp.float32) \
        + bvt_ref[0]

    at = jax.lax.dot_general(
        k, qt, (((1,), (0,)), ((), ())),
        preferred_element_type=jnp.float32) * SCALE

    p = _topk_weights(at)

    n = at.shape[0]
    v1t = jnp.concatenate([vt, jnp.ones((1, n), jnp.float32)], axis=0)
    apt = jax.lax.dot_general(
        v1t, p, (((1,), (0,)), ((), ())),
        preferred_element_type=jnp.float32)
    attnout_t = apt[:HEAD_DIM] / apt[HEAD_DIM:HEAD_DIM + 1]

    ao_ref[pl.ds(h * HEAD_DIM, HEAD_DIM), :] = attnout_t

    @pl.when(h == NUM_HEADS - 1)
    def _():
        out_t = jnp.dot(wpt_ref[...], ao_ref[...],
                        preferred_element_type=jnp.float32)
        out_ref[0] = out_t.T + bp_ref[0]


@jax.jit
def kernel(x, Wq, bq, Wkv, bkv, Wp, bp):
    B, N, C = x.shape
    H, dh = NUM_HEADS, HEAD_DIM
    xt = x.transpose(0, 2, 1)
    Wqt_r = Wq.reshape(C, H, dh).transpose(1, 2, 0)
    Wk_r = Wkv[:, :C].reshape(C, H, dh).transpose(1, 0, 2)
    Wvt_r = Wkv[:, C:].reshape(C, H, dh).transpose(1, 2, 0)
    bqt_r = bq.reshape(H, dh, 1)
    bk_r = bkv[:C].reshape(H, 1, dh)
    bvt_r = bkv[C:].reshape(H, dh, 1)
    bp_r = bp.reshape(1, C)

    grid = (B, H)
    out = pl.pallas_call(
        _fused_attn_kernel,
        grid=grid,
        in_specs=[
            pl.BlockSpec((1, N, C), lambda b, h: (b, 0, 0)),
            pl.BlockSpec((1, C, N), lambda b, h: (b, 0, 0)),
            pl.BlockSpec((1, dh, C), lambda b, h: (h, 0, 0)),
            pl.BlockSpec((1, dh, 1), lambda b, h: (h, 0, 0)),
            pl.BlockSpec((1, C, dh), lambda b, h: (h, 0, 0)),
            pl.BlockSpec((1, 1, dh), lambda b, h: (h, 0, 0)),
            pl.BlockSpec((1, dh, C), lambda b, h: (h, 0, 0)),
            pl.BlockSpec((1, dh, 1), lambda b, h: (h, 0, 0)),
            pl.BlockSpec((C, C), lambda b, h: (0, 0)),
            pl.BlockSpec((1, C), lambda b, h: (0, 0)),
        ],
        out_specs=pl.BlockSpec((1, N, C), lambda b, h: (b, 0, 0)),
        out_shape=jax.ShapeDtypeStruct((B, N, C), jnp.float32),
        scratch_shapes=[pltpu.VMEM((C, N), jnp.float32)],
    )(x, xt, Wqt_r, bqt_r, Wk_r, bk_r, Wvt_r, bvt_r, Wp.T, bp_r)
    return out


# speedup vs baseline: 50.5633x; 1.0244x over previous
import jax
import jax.numpy as jnp
from jax.experimental import pallas as pl
from jax.experimental.pallas import tpu as pltpu

DIM = 384
NUM_HEADS = 8
TOP_K = 16
HEAD_DIM = DIM // NUM_HEADS
SCALE = HEAD_DIM ** (-0.5)

_SORT16 = (
    [(0, 1), (2, 3), (4, 5), (6, 7), (8, 9), (10, 11), (12, 13), (14, 15)]
    + [(0, 2), (1, 3), (4, 6), (5, 7), (8, 10), (9, 11), (12, 14), (13, 15)]
    + [(0, 4), (1, 5), (2, 6), (3, 7), (8, 12), (9, 13), (10, 14), (11, 15)]
    + [(0, 8), (1, 9), (2, 10), (3, 11), (4, 12), (5, 13), (6, 14), (7, 15)]
    + [(5, 10), (6, 9), (3, 12), (13, 14), (7, 11), (1, 2), (4, 8)]
    + [(1, 4), (7, 13), (2, 8), (11, 14), (5, 6), (9, 10)]
    + [(2, 4), (11, 13), (3, 8), (7, 12)]
    + [(6, 8), (10, 12), (3, 5), (7, 9)]
    + [(3, 4), (5, 6), (7, 8), (9, 10), (11, 12)]
    + [(6, 7), (8, 9)])


def _ce(slots, i, j):
    a, b = slots[i], slots[j]
    slots[i] = jnp.maximum(a, b)
    slots[j] = jnp.minimum(a, b)


def _bitonic_resort_desc(slots):
    n = len(slots)
    stride = n // 2
    while stride >= 1:
        for i in range(n):
            if (i % (2 * stride)) < stride:
                _ce(slots, i, i + stride)
        stride //= 2
    return slots


def _topk_weights(at):
    rows, cols = at.shape
    runs = rows // 16
    parts = []
    for c0 in range(0, cols, 128):
        ch = at[:, c0:c0 + 128]
        t = [ch[runs * s:runs * (s + 1), :] for s in range(16)]
        for i, j in _SORT16:
            _ce(t, i, j)
        n = runs
        while n > 2:
            m = n // 2
            x = [s[:m] for s in t]
            y = [s[m:] for s in t]
            t = _bitonic_resort_desc(
                [jnp.maximum(x[i], y[15 - i]) for i in range(16)])
            n = m
        z = [jnp.maximum(t[i][:1], t[15 - i][1:]) for i in range(16)]
        thr = z[0]
        for zi in z[1:]:
            thr = jnp.minimum(thr, zi)
        parts.append(jnp.exp(jnp.where(ch >= thr, ch, 0.0)))
    return jnp.concatenate(parts, axis=1)


def _fused_attn_kernel(x_ref, wqt_ref, bqt_ref, wk_ref, bk_ref,
                       wvt_ref, bvt_ref, wpt_ref, bp_ref, out_ref, ao_ref):
    h = pl.program_id(1)
    xb = x_ref[0]

    qt = jax.lax.dot_general(
        wqt_ref[0], xb, (((1,), (1,)), ((), ())),
        preferred_element_type=jnp.float32) + bqt_ref[0]
    qt = jnp.maximum(qt, 0.0)
    k = jnp.dot(xb, wk_ref[0], preferred_element_type=jnp.float32) + bk_ref[0]
    vt = jax.lax.dot_general(
        wvt_ref[0], xb, (((1,), (1,)), ((), ())),
        preferred_element_type=jnp.float32) + bvt_ref[0]

    at = jax.lax.dot_general(
        k, qt, (((1,), (0,)), ((), ())),
        preferred_element_type=jnp.float32) * SCALE

    p = _topk_weights(at)

    n = at.shape[0]
    v1t = jnp.concatenate([vt, jnp.ones((1, n), jnp.float32)], axis=0)
    apt = jax.lax.dot_general(
        v1t, p, (((1,), (0,)), ((), ())),
        preferred_element_type=jnp.float32)
    attnout_t = apt[:HEAD_DIM] / apt[HEAD_DIM:HEAD_DIM + 1]

    ao_ref[pl.ds(h * HEAD_DIM, HEAD_DIM), :] = attnout_t

    @pl.when(h == NUM_HEADS - 1)
    def _():
        out_t = jnp.dot(wpt_ref[...], ao_ref[...],
                        preferred_element_type=jnp.float32)
        out_ref[0] = out_t.T + bp_ref[0]


@jax.jit
def kernel(x, Wq, bq, Wkv, bkv, Wp, bp):
    B, N, C = x.shape
    H, dh = NUM_HEADS, HEAD_DIM
    Wqt_r = Wq.reshape(C, H, dh).transpose(1, 2, 0)
    Wk_r = Wkv[:, :C].reshape(C, H, dh).transpose(1, 0, 2)
    Wvt_r = Wkv[:, C:].reshape(C, H, dh).transpose(1, 2, 0)
    bqt_r = bq.reshape(H, dh, 1)
    bk_r = bkv[:C].reshape(H, 1, dh)
    bvt_r = bkv[C:].reshape(H, dh, 1)
    bp_r = bp.reshape(1, C)

    grid = (B, H)
    out = pl.pallas_call(
        _fused_attn_kernel,
        grid=grid,
        in_specs=[
            pl.BlockSpec((1, N, C), lambda b, h: (b, 0, 0)),
            pl.BlockSpec((1, dh, C), lambda b, h: (h, 0, 0)),
            pl.BlockSpec((1, dh, 1), lambda b, h: (h, 0, 0)),
            pl.BlockSpec((1, C, dh), lambda b, h: (h, 0, 0)),
            pl.BlockSpec((1, 1, dh), lambda b, h: (h, 0, 0)),
            pl.BlockSpec((1, dh, C), lambda b, h: (h, 0, 0)),
            pl.BlockSpec((1, dh, 1), lambda b, h: (h, 0, 0)),
            pl.BlockSpec((C, C), lambda b, h: (0, 0)),
            pl.BlockSpec((1, C), lambda b, h: (0, 0)),
        ],
        out_specs=pl.BlockSpec((1, N, C), lambda b, h: (b, 0, 0)),
        out_shape=jax.ShapeDtypeStruct((B, N, C), jnp.float32),
        scratch_shapes=[pltpu.VMEM((C, N), jnp.float32)],
    )(x, Wqt_r, bqt_r, Wk_r, bk_r, Wvt_r, bvt_r, Wp.T, bp_r)
    return out


# speedup vs baseline: 52.6550x; 1.0414x over previous
import jax
import jax.numpy as jnp
from jax.experimental import pallas as pl
from jax.experimental.pallas import tpu as pltpu

DIM = 384
NUM_HEADS = 8
TOP_K = 16
HEAD_DIM = DIM // NUM_HEADS
SCALE = HEAD_DIM ** (-0.5)

_SORT16 = (
    [(0, 1), (2, 3), (4, 5), (6, 7), (8, 9), (10, 11), (12, 13), (14, 15)]
    + [(0, 2), (1, 3), (4, 6), (5, 7), (8, 10), (9, 11), (12, 14), (13, 15)]
    + [(0, 4), (1, 5), (2, 6), (3, 7), (8, 12), (9, 13), (10, 14), (11, 15)]
    + [(0, 8), (1, 9), (2, 10), (3, 11), (4, 12), (5, 13), (6, 14), (7, 15)]
    + [(5, 10), (6, 9), (3, 12), (13, 14), (7, 11), (1, 2), (4, 8)]
    + [(1, 4), (7, 13), (2, 8), (11, 14), (5, 6), (9, 10)]
    + [(2, 4), (11, 13), (3, 8), (7, 12)]
    + [(6, 8), (10, 12), (3, 5), (7, 9)]
    + [(3, 4), (5, 6), (7, 8), (9, 10), (11, 12)]
    + [(6, 7), (8, 9)])


def _ce(slots, i, j):
    a, b = slots[i], slots[j]
    slots[i] = jnp.maximum(a, b)
    slots[j] = jnp.minimum(a, b)


def _bitonic_resort_desc(slots):
    n = len(slots)
    stride = n // 2
    while stride >= 1:
        for i in range(n):
            if (i % (2 * stride)) < stride:
                _ce(slots, i, i + stride)
        stride //= 2
    return slots


def _topk_weights(at):
    rows, cols = at.shape
    runs = rows // 16
    parts = []
    for c0 in range(0, cols, 128):
        ch = at[:, c0:c0 + 128]
        t = [ch[runs * s:runs * (s + 1), :] for s in range(16)]
        for i, j in _SORT16:
            _ce(t, i, j)
        n = runs
        while n > 2:
            m = n // 2
            x = [s[:m] for s in t]
            y = [s[m:] for s in t]
            t = _bitonic_resort_desc(
                [jnp.maximum(x[i], y[15 - i]) for i in range(16)])
            n = m
        z = [jnp.maximum(t[i][:1], t[15 - i][1:]) for i in range(16)]
        thr = z[0]
        for zi in z[1:]:
            thr = jnp.minimum(thr, zi)
        parts.append(jnp.exp(jnp.where(ch >= thr, ch, 0.0)))
    return jnp.concatenate(parts, axis=1)


_HEADS_PER_PROG = 2


def _fused_attn_kernel(x_ref, wqt_ref, bqt_ref, wk_ref, bk_ref,
                       wvt_ref, bvt_ref, wpt_ref, bp_ref, out_ref, ao_ref):
    g = pl.program_id(1)
    xb = x_ref[0]

    for hh in range(_HEADS_PER_PROG):
        qt = jax.lax.dot_general(
            wqt_ref[hh], xb, (((1,), (1,)), ((), ())),
            preferred_element_type=jnp.float32) + bqt_ref[hh]
        qt = jnp.maximum(qt, 0.0)
        k = jnp.dot(xb, wk_ref[hh],
                    preferred_element_type=jnp.float32) + bk_ref[hh]
        vt = jax.lax.dot_general(
            wvt_ref[hh], xb, (((1,), (1,)), ((), ())),
            preferred_element_type=jnp.float32) + bvt_ref[hh]

        at = jax.lax.dot_general(
            k, qt, (((1,), (0,)), ((), ())),
            preferred_element_type=jnp.float32) * SCALE

        p = _topk_weights(at)

        n = at.shape[0]
        v1t = jnp.concatenate([vt, jnp.ones((1, n), jnp.float32)], axis=0)
        apt = jax.lax.dot_general(
            v1t, p, (((1,), (0,)), ((), ())),
            preferred_element_type=jnp.float32)
        attnout_t = apt[:HEAD_DIM] / apt[HEAD_DIM:HEAD_DIM + 1]

        ao_ref[pl.ds((g * _HEADS_PER_PROG + hh) * HEAD_DIM, HEAD_DIM), :] = \
            attnout_t

    @pl.when(g == NUM_HEADS // _HEADS_PER_PROG - 1)
    def _():
        out_t = jnp.dot(wpt_ref[...], ao_ref[...],
                        preferred_element_type=jnp.float32)
        out_ref[0] = out_t.T + bp_ref[0]


@jax.jit
def kernel(x, Wq, bq, Wkv, bkv, Wp, bp):
    B, N, C = x.shape
    H, dh = NUM_HEADS, HEAD_DIM
    Wqt_r = Wq.reshape(C, H, dh).transpose(1, 2, 0)
    Wk_r = Wkv[:, :C].reshape(C, H, dh).transpose(1, 0, 2)
    Wvt_r = Wkv[:, C:].reshape(C, H, dh).transpose(1, 2, 0)
    bqt_r = bq.reshape(H, dh, 1)
    bk_r = bkv[:C].reshape(H, 1, dh)
    bvt_r = bkv[C:].reshape(H, dh, 1)
    bp_r = bp.reshape(1, C)

    hp = _HEADS_PER_PROG
    grid = (B, H // hp)
    out = pl.pallas_call(
        _fused_attn_kernel,
        grid=grid,
        in_specs=[
            pl.BlockSpec((1, N, C), lambda b, g: (b, 0, 0)),
            pl.BlockSpec((hp, dh, C), lambda b, g: (g, 0, 0)),
            pl.BlockSpec((hp, dh, 1), lambda b, g: (g, 0, 0)),
            pl.BlockSpec((hp, C, dh), lambda b, g: (g, 0, 0)),
            pl.BlockSpec((hp, 1, dh), lambda b, g: (g, 0, 0)),
            pl.BlockSpec((hp, dh, C), lambda b, g: (g, 0, 0)),
            pl.BlockSpec((hp, dh, 1), lambda b, g: (g, 0, 0)),
            pl.BlockSpec((C, C), lambda b, g: (0, 0)),
            pl.BlockSpec((1, C), lambda b, g: (0, 0)),
        ],
        out_specs=pl.BlockSpec((1, N, C), lambda b, g: (b, 0, 0)),
        out_shape=jax.ShapeDtypeStruct((B, N, C), jnp.float32),
        scratch_shapes=[pltpu.VMEM((C, N), jnp.float32)],
    )(x, Wqt_r, bqt_r, Wk_r, bk_r, Wvt_r, bvt_r, Wp.T, bp_r)
    return out


# speedup vs baseline: 53.4657x; 1.0154x over previous
import jax
import jax.numpy as jnp
from jax.experimental import pallas as pl
from jax.experimental.pallas import tpu as pltpu

DIM = 384
NUM_HEADS = 8
TOP_K = 16
HEAD_DIM = DIM // NUM_HEADS
SCALE = HEAD_DIM ** (-0.5)

_SORT16 = (
    [(0, 1), (2, 3), (4, 5), (6, 7), (8, 9), (10, 11), (12, 13), (14, 15)]
    + [(0, 2), (1, 3), (4, 6), (5, 7), (8, 10), (9, 11), (12, 14), (13, 15)]
    + [(0, 4), (1, 5), (2, 6), (3, 7), (8, 12), (9, 13), (10, 14), (11, 15)]
    + [(0, 8), (1, 9), (2, 10), (3, 11), (4, 12), (5, 13), (6, 14), (7, 15)]
    + [(5, 10), (6, 9), (3, 12), (13, 14), (7, 11), (1, 2), (4, 8)]
    + [(1, 4), (7, 13), (2, 8), (11, 14), (5, 6), (9, 10)]
    + [(2, 4), (11, 13), (3, 8), (7, 12)]
    + [(6, 8), (10, 12), (3, 5), (7, 9)]
    + [(3, 4), (5, 6), (7, 8), (9, 10), (11, 12)]
    + [(6, 7), (8, 9)])


def _ce(slots, i, j):
    a, b = slots[i], slots[j]
    slots[i] = jnp.maximum(a, b)
    slots[j] = jnp.minimum(a, b)


def _bitonic_resort_desc(slots):
    n = len(slots)
    stride = n // 2
    while stride >= 1:
        for i in range(n):
            if (i % (2 * stride)) < stride:
                _ce(slots, i, i + stride)
        stride //= 2
    return slots


def _topk_weights(at):
    rows, cols = at.shape
    runs = rows // 16
    parts = []
    for c0 in range(0, cols, 128):
        ch = at[:, c0:c0 + 128]
        t = [ch[runs * s:runs * (s + 1), :] for s in range(16)]
        for i, j in _SORT16:
            _ce(t, i, j)
        n = runs
        while n > 2:
            m = n // 2
            x = [s[:m] for s in t]
            y = [s[m:] for s in t]
            t = _bitonic_resort_desc(
                [jnp.maximum(x[i], y[15 - i]) for i in range(16)])
            n = m
        z = [jnp.maximum(t[i][:1], t[15 - i][1:]) for i in range(16)]
        thr = z[0]
        for zi in z[1:]:
            thr = jnp.minimum(thr, zi)
        parts.append(jnp.exp(jnp.where(ch >= thr, ch, 0.0)))
    return jnp.concatenate(parts, axis=1)


_HEADS_PER_PROG = 4


def _fused_attn_kernel(x_ref, wqt_ref, bqt_ref, wk_ref, bk_ref,
                       wvt_ref, bvt_ref, wpt_ref, bp_ref, out_ref, ao_ref):
    g = pl.program_id(1)
    xb = x_ref[0]

    for hh in range(_HEADS_PER_PROG):
        qt = jax.lax.dot_general(
            wqt_ref[hh], xb, (((1,), (1,)), ((), ())),
            preferred_element_type=jnp.float32) + bqt_ref[hh]
        qt = jnp.maximum(qt, 0.0)
        k = jnp.dot(xb, wk_ref[hh],
                    preferred_element_type=jnp.float32) + bk_ref[hh]
        vt = jax.lax.dot_general(
            wvt_ref[hh], xb, (((1,), (1,)), ((), ())),
            preferred_element_type=jnp.float32) + bvt_ref[hh]

        at = jax.lax.dot_general(
            k, qt, (((1,), (0,)), ((), ())),
            preferred_element_type=jnp.float32) * SCALE

        p = _topk_weights(at)

        n = at.shape[0]
        v1t = jnp.concatenate([vt, jnp.ones((1, n), jnp.float32)], axis=0)
        apt = jax.lax.dot_general(
            v1t, p, (((1,), (0,)), ((), ())),
            preferred_element_type=jnp.float32)
        attnout_t = apt[:HEAD_DIM] / apt[HEAD_DIM:HEAD_DIM + 1]

        ao_ref[pl.ds((g * _HEADS_PER_PROG + hh) * HEAD_DIM, HEAD_DIM), :] = \
            attnout_t

    @pl.when(g == NUM_HEADS // _HEADS_PER_PROG - 1)
    def _():
        out_t = jnp.dot(wpt_ref[...], ao_ref[...],
                        preferred_element_type=jnp.float32)
        out_ref[0] = out_t.T + bp_ref[0]


@jax.jit
def kernel(x, Wq, bq, Wkv, bkv, Wp, bp):
    B, N, C = x.shape
    H, dh = NUM_HEADS, HEAD_DIM
    Wqt_r = Wq.reshape(C, H, dh).transpose(1, 2, 0)
    Wk_r = Wkv[:, :C].reshape(C, H, dh).transpose(1, 0, 2)
    Wvt_r = Wkv[:, C:].reshape(C, H, dh).transpose(1, 2, 0)
    bqt_r = bq.reshape(H, dh, 1)
    bk_r = bkv[:C].reshape(H, 1, dh)
    bvt_r = bkv[C:].reshape(H, dh, 1)
    bp_r = bp.reshape(1, C)

    hp = _HEADS_PER_PROG
    grid = (B, H // hp)
    out = pl.pallas_call(
        _fused_attn_kernel,
        grid=grid,
        in_specs=[
            pl.BlockSpec((1, N, C), lambda b, g: (b, 0, 0)),
            pl.BlockSpec((hp, dh, C), lambda b, g: (g, 0, 0)),
            pl.BlockSpec((hp, dh, 1), lambda b, g: (g, 0, 0)),
            pl.BlockSpec((hp, C, dh), lambda b, g: (g, 0, 0)),
            pl.BlockSpec((hp, 1, dh), lambda b, g: (g, 0, 0)),
            pl.BlockSpec((hp, dh, C), lambda b, g: (g, 0, 0)),
            pl.BlockSpec((hp, dh, 1), lambda b, g: (g, 0, 0)),
            pl.BlockSpec((C, C), lambda b, g: (0, 0)),
            pl.BlockSpec((1, C), lambda b, g: (0, 0)),
        ],
        out_specs=pl.BlockSpec((1, N, C), lambda b, g: (b, 0, 0)),
        out_shape=jax.ShapeDtypeStruct((B, N, C), jnp.float32),
        scratch_shapes=[pltpu.VMEM((C, N), jnp.float32)],
    )(x, Wqt_r, bqt_r, Wk_r, bk_r, Wvt_r, bvt_r, Wp.T, bp_r)
    return out
